# baseline (speedup 1.0000x reference)
import functools

import jax
import jax.numpy as jnp
from jax import lax
from jax.experimental import pallas as pl
from jax.experimental.pallas import tpu as pltpu
from jax.experimental.pallas import tpu_sc as plsc

B, L, V, D, F, U, NCLS = 4096, 200, 100000, 128, 128, 512, 5
TB = 64
RB = TB * L

NC, NS = 2, 16
NW = NC * NS
ROWS = B * L
CH = 64
HB = 4
NBUF = 2 * HB


def _sc_gather_body(rpw, nchunk, emb_hbm, idx_hbm, out_hbm,
                    idx_v, rows_v, gsem, osem0, osem1):
    wid = lax.axis_index("s") * NC + lax.axis_index("c")
    base = pl.multiple_of(wid * rpw, rpw)
    pltpu.sync_copy(idx_hbm.at[pl.ds(base, rpw)], idx_v)
    osems = (osem0, osem1)

    @pl.loop(0, nchunk, step=NBUF)
    def _grp(g):
        for h in range(2):
            osem = osems[h]

            @pl.when(g > 0)
            def _drain():
                for b in range(HB):
                    pltpu.make_async_copy(
                        rows_v.at[h * HB + b],
                        out_hbm.at[pl.ds(base, CH)], osem).wait()

            gds = []
            for b in range(HB):
                off = pl.multiple_of((g + h * HB + b) * CH, CH)
                gd = pltpu.make_async_copy(
                    emb_hbm.at[idx_v.at[pl.ds(off, CH)]],
                    rows_v.at[h * HB + b], gsem)
                gd.start()
                gds.append((gd, off))
            for b in range(HB):
                gd, off = gds[b]
                gd.wait()
                pltpu.make_async_copy(
                    rows_v.at[h * HB + b],
                    out_hbm.at[pl.ds(base + off, CH)], osem).start()

    for h in range(2):
        for b in range(HB):
            pltpu.make_async_copy(
                rows_v.at[h * HB + b],
                out_hbm.at[pl.ds(base, CH)], osems[h]).wait()


def _sc_gather(emb, idx_flat):
    rows = idx_flat.shape[0]
    rpw = rows // NW
    nchunk = rpw // CH
    mesh = plsc.VectorSubcoreMesh(core_axis_name="c", subcore_axis_name="s")
    return pl.kernel(
        functools.partial(_sc_gather_body, rpw, nchunk),
        out_type=jax.ShapeDtypeStruct((rows, D), jnp.float32),
        mesh=mesh,
        scratch_types=[
            pltpu.VMEM((rpw,), jnp.int32),
            pltpu.VMEM((NBUF, CH, D), jnp.float32),
            pltpu.SemaphoreType.DMA,
            pltpu.SemaphoreType.DMA,
            pltpu.SemaphoreType.DMA,
        ],
    )(emb, idx_flat)


def _tc_body(x_ref, wq_ref, w2c_ref, b1_ref, b2_ref, b3_ref,
             wd_ref, bd_ref, wlp_ref, blp_ref, out_ref):
    xb = x_ref[...].astype(jnp.bfloat16)
    xs = jnp.concatenate([xb[1:RB, :], xb[RB - 1:RB, :]], axis=0)
    xx = jnp.concatenate([xb, xs], axis=1)
    q2 = jnp.dot(xx, wq_ref[...], preferred_element_type=jnp.float32)
    q = q2.reshape(TB, L, 512)
    r2 = jnp.dot(xb, w2c_ref[...], preferred_element_type=jnp.float32)
    r3 = r2.reshape(TB, L, F)

    s1 = q[:, 0:199, 0:128] + b1_ref[...]
    s2 = q[:, 0:198, 128:256] + r3[:, 2:200, :] + b2_ref[...]
    s3 = (q[:, 0:197, 256:384] + q[:, 2:199, 384:512]
          + b3_ref[...])
    m1 = jnp.max(jnp.maximum(s1, 0.0), axis=1)
    m2 = jnp.max(jnp.maximum(s2, 0.0), axis=1)
    m3 = jnp.max(jnp.maximum(s3, 0.0), axis=1)
    cat = jnp.concatenate([m1, m2, m3], axis=1)

    h = jnp.maximum(
        jnp.dot(cat, wd_ref[...], preferred_element_type=jnp.float32)
        + bd_ref[...], 0.0)
    logits = (jnp.dot(h, wlp_ref[...], preferred_element_type=jnp.float32)
              + blp_ref[...])
    mx = jnp.max(logits, axis=1, keepdims=True)
    e = jnp.exp(logits - mx)
    out_ref[...] = e / jnp.sum(e, axis=1, keepdims=True)


def _tc_forward(x2d, wq, w2c, b1r, b2r, b3r, wd, bdr, wlp, blp):
    nb = x2d.shape[0] // RB
    return pl.pallas_call(
        _tc_body,
        grid=(nb,),
        in_specs=[
            pl.BlockSpec((RB, D), lambda i: (i, 0)),
            pl.BlockSpec((2 * D, 512), lambda i: (0, 0)),
            pl.BlockSpec((D, F), lambda i: (0, 0)),
            pl.BlockSpec((1, 1, F), lambda i: (0, 0, 0)),
            pl.BlockSpec((1, 1, F), lambda i: (0, 0, 0)),
            pl.BlockSpec((1, 1, F), lambda i: (0, 0, 0)),
            pl.BlockSpec((3 * F, U), lambda i: (0, 0)),
            pl.BlockSpec((1, U), lambda i: (0, 0)),
            pl.BlockSpec((U, 128), lambda i: (0, 0)),
            pl.BlockSpec((1, 128), lambda i: (0, 0)),
        ],
        out_specs=pl.BlockSpec((TB, 128), lambda i: (i, 0)),
        out_shape=jax.ShapeDtypeStruct((nb * TB, 128), jnp.float32),
    )(x2d, wq, w2c, b1r, b2r, b3r, wd, bdr, wlp, blp)


NSLICE = 1


def kernel(inputs, emb, W1, b1, W2, b2, W3, b3, Wd, bd, Wl, bl, training):
    idx_flat = inputs.reshape(B * L).astype(jnp.int32)

    wq = jnp.concatenate([
        W1.reshape(2 * D, F),
        W2[0:2].reshape(2 * D, F),
        W3[0:2].reshape(2 * D, F),
        W3[2:4].reshape(2 * D, F),
    ], axis=1).astype(jnp.bfloat16)
    w2c = W2[2].astype(jnp.bfloat16)

    b1r = b1.reshape(1, 1, F)
    b2r = b2.reshape(1, 1, F)
    b3r = b3.reshape(1, 1, F)
    bdr = bd.reshape(1, U)
    wlp = jnp.pad(Wl, ((0, 0), (0, 128 - NCLS)))
    blp = jnp.concatenate(
        [bl, jnp.full((128 - NCLS,), -1e30, dtype=jnp.float32)]).reshape(1, 128)

    srows = ROWS // NSLICE
    outs = []
    for s in range(NSLICE):
        idx_s = lax.slice(idx_flat, (s * srows,), ((s + 1) * srows,))
        x_s = _sc_gather(emb, idx_s)
        outs.append(_tc_forward(x_s, wq, w2c, b1r, b2r, b3r, Wd, bdr,
                                wlp, blp))
    out = jnp.concatenate(outs, axis=0)
    return out[:, :NCLS]


# speedup vs baseline: 1.0097x; 1.0097x over previous
import functools

import jax
import jax.numpy as jnp
from jax import lax
from jax.experimental import pallas as pl
from jax.experimental.pallas import tpu as pltpu
from jax.experimental.pallas import tpu_sc as plsc

B, L, V, D, F, U, NCLS = 4096, 200, 100000, 128, 128, 512, 5
TB = 64
RB = TB * L

NC, NS = 2, 16
NW = NC * NS
ROWS = B * L
CH = 64
HB = 4
NBUF = 2 * HB


def _sc_gather_body(rpw, nchunk, emb_hbm, idx_hbm, out_hbm,
                    idx_v, rows_v, gsem, osem0, osem1):
    wid = lax.axis_index("s") * NC + lax.axis_index("c")
    base = pl.multiple_of(wid * rpw, rpw)
    pltpu.sync_copy(idx_hbm.at[pl.ds(base, rpw)], idx_v)
    osems = (osem0, osem1)

    @pl.loop(0, nchunk, step=NBUF)
    def _grp(g):
        for h in range(2):
            osem = osems[h]

            @pl.when(g > 0)
            def _drain():
                for b in range(HB):
                    pltpu.make_async_copy(
                        rows_v.at[h * HB + b],
                        out_hbm.at[pl.ds(base, CH)], osem).wait()

            gds = []
            for b in range(HB):
                off = pl.multiple_of((g + h * HB + b) * CH, CH)
                gd = pltpu.make_async_copy(
                    emb_hbm.at[idx_v.at[pl.ds(off, CH)]],
                    rows_v.at[h * HB + b], gsem)
                gd.start()
                gds.append((gd, off))
            for b in range(HB):
                gd, off = gds[b]
                gd.wait()
                pltpu.make_async_copy(
                    rows_v.at[h * HB + b],
                    out_hbm.at[pl.ds(base + off, CH)], osem).start()

    for h in range(2):
        for b in range(HB):
            pltpu.make_async_copy(
                rows_v.at[h * HB + b],
                out_hbm.at[pl.ds(base, CH)], osems[h]).wait()


def _sc_gather(emb, idx_flat):
    rows = idx_flat.shape[0]
    rpw = rows // NW
    nchunk = rpw // CH
    mesh = plsc.VectorSubcoreMesh(core_axis_name="c", subcore_axis_name="s")
    return pl.kernel(
        functools.partial(_sc_gather_body, rpw, nchunk),
        out_type=jax.ShapeDtypeStruct((rows, D), jnp.float32),
        mesh=mesh,
        scratch_types=[
            pltpu.VMEM((rpw,), jnp.int32),
            pltpu.VMEM((NBUF, CH, D), jnp.float32),
            pltpu.SemaphoreType.DMA,
            pltpu.SemaphoreType.DMA,
            pltpu.SemaphoreType.DMA,
        ],
    )(emb, idx_flat)


def _tc_body(x_ref, wq_ref, w2c_ref, b1_ref, b2_ref, b3_ref,
             wd_ref, bd_ref, wlp_ref, blp_ref, out_ref):
    xb = x_ref[...].astype(jnp.bfloat16)
    xs = jnp.concatenate([xb[1:RB, :], xb[RB - 1:RB, :]], axis=0)
    xx = jnp.concatenate([xb, xs], axis=1)
    q2 = jnp.dot(xx, wq_ref[...],
                 preferred_element_type=jnp.float32).astype(jnp.bfloat16)
    q = q2.reshape(TB, L, 512)
    r2 = jnp.dot(xb, w2c_ref[...],
                 preferred_element_type=jnp.float32).astype(jnp.bfloat16)
    r3 = r2.reshape(TB, L, F)

    zero = jnp.bfloat16(0)
    s1 = q[:, 0:199, 0:128] + b1_ref[...]
    s2 = q[:, 0:198, 128:256] + r3[:, 2:200, :] + b2_ref[...]
    s3 = (q[:, 0:197, 256:384] + q[:, 2:199, 384:512]
          + b3_ref[...])
    m1 = jnp.max(jnp.maximum(s1, zero), axis=1)
    m2 = jnp.max(jnp.maximum(s2, zero), axis=1)
    m3 = jnp.max(jnp.maximum(s3, zero), axis=1)
    cat = jnp.concatenate([m1, m2, m3], axis=1).astype(jnp.float32)

    h = jnp.maximum(
        jnp.dot(cat, wd_ref[...], preferred_element_type=jnp.float32)
        + bd_ref[...], 0.0)
    logits = (jnp.dot(h, wlp_ref[...], preferred_element_type=jnp.float32)
              + blp_ref[...])
    mx = jnp.max(logits, axis=1, keepdims=True)
    e = jnp.exp(logits - mx)
    out_ref[...] = e / jnp.sum(e, axis=1, keepdims=True)


def _tc_forward(x2d, wq, w2c, b1r, b2r, b3r, wd, bdr, wlp, blp):
    nb = x2d.shape[0] // RB
    return pl.pallas_call(
        _tc_body,
        grid=(nb,),
        in_specs=[
            pl.BlockSpec((RB, D), lambda i: (i, 0)),
            pl.BlockSpec((2 * D, 512), lambda i: (0, 0)),
            pl.BlockSpec((D, F), lambda i: (0, 0)),
            pl.BlockSpec((1, 1, F), lambda i: (0, 0, 0)),
            pl.BlockSpec((1, 1, F), lambda i: (0, 0, 0)),
            pl.BlockSpec((1, 1, F), lambda i: (0, 0, 0)),
            pl.BlockSpec((3 * F, U), lambda i: (0, 0)),
            pl.BlockSpec((1, U), lambda i: (0, 0)),
            pl.BlockSpec((U, 128), lambda i: (0, 0)),
            pl.BlockSpec((1, 128), lambda i: (0, 0)),
        ],
        out_specs=pl.BlockSpec((TB, 128), lambda i: (i, 0)),
        out_shape=jax.ShapeDtypeStruct((nb * TB, 128), jnp.float32),
    )(x2d, wq, w2c, b1r, b2r, b3r, wd, bdr, wlp, blp)


NSLICE = 2


def kernel(inputs, emb, W1, b1, W2, b2, W3, b3, Wd, bd, Wl, bl, training):
    idx_flat = inputs.reshape(B * L).astype(jnp.int32)

    wq = jnp.concatenate([
        W1.reshape(2 * D, F),
        W2[0:2].reshape(2 * D, F),
        W3[0:2].reshape(2 * D, F),
        W3[2:4].reshape(2 * D, F),
    ], axis=1).astype(jnp.bfloat16)
    w2c = W2[2].astype(jnp.bfloat16)

    b1r = b1.reshape(1, 1, F).astype(jnp.bfloat16)
    b2r = b2.reshape(1, 1, F).astype(jnp.bfloat16)
    b3r = b3.reshape(1, 1, F).astype(jnp.bfloat16)
    bdr = bd.reshape(1, U)
    wlp = jnp.pad(Wl, ((0, 0), (0, 128 - NCLS)))
    blp = jnp.concatenate(
        [bl, jnp.full((128 - NCLS,), -1e30, dtype=jnp.float32)]).reshape(1, 128)

    srows = ROWS // NSLICE
    outs = []
    x_prev = None
    for s in range(NSLICE):
        idx_s = lax.slice(idx_flat, (s * srows,), ((s + 1) * srows,))
        if x_prev is not None:
            idx_s, _ = lax.optimization_barrier((idx_s, x_prev))
        x_s = _sc_gather(emb, idx_s)
        x_prev = x_s
        outs.append(_tc_forward(x_s, wq, w2c, b1r, b2r, b3r, Wd, bdr,
                                wlp, blp))
    out = outs[0] if NSLICE == 1 else jnp.concatenate(outs, axis=0)
    return out[:, :NCLS]


# speedup vs baseline: 1.1692x; 1.1581x over previous
import functools

import jax
import jax.numpy as jnp
from jax import lax
from jax.experimental import pallas as pl
from jax.experimental.pallas import tpu as pltpu
from jax.experimental.pallas import tpu_sc as plsc

B, L, V, D, F, U, NCLS = 4096, 200, 100000, 128, 128, 512, 5
TB = 64
RB = TB * L

NC, NS = 2, 16
NW = NC * NS
ROWS = B * L
CH = 64
HB = 4
NBUF = 2 * HB


def _sc_gather_body(rpw, nchunk, emb_hbm, idx_hbm, out_hbm,
                    idx_v, rows_v, gsem, osem0, osem1):
    wid = lax.axis_index("s") * NC + lax.axis_index("c")
    base = pl.multiple_of(wid * rpw, rpw)
    pltpu.sync_copy(idx_hbm.at[pl.ds(base, rpw)], idx_v)
    osems = (osem0, osem1)

    @pl.loop(0, nchunk, step=NBUF)
    def _grp(g):
        for h in range(2):
            osem = osems[h]

            @pl.when(g > 0)
            def _drain():
                for b in range(HB):
                    pltpu.make_async_copy(
                        rows_v.at[h * HB + b],
                        out_hbm.at[pl.ds(base, CH)], osem).wait()

            gds = []
            for b in range(HB):
                off = pl.multiple_of((g + h * HB + b) * CH, CH)
                gd = pltpu.make_async_copy(
                    emb_hbm.at[idx_v.at[pl.ds(off, CH)]],
                    rows_v.at[h * HB + b], gsem)
                gd.start()
                gds.append((gd, off))
            for b in range(HB):
                gd, off = gds[b]
                gd.wait()
                pltpu.make_async_copy(
                    rows_v.at[h * HB + b],
                    out_hbm.at[pl.ds(base + off, CH)], osem).start()

    for h in range(2):
        for b in range(HB):
            pltpu.make_async_copy(
                rows_v.at[h * HB + b],
                out_hbm.at[pl.ds(base, CH)], osems[h]).wait()


def _sc_gather(emb, idx_flat):
    rows = idx_flat.shape[0]
    rpw = rows // NW
    nchunk = rpw // CH
    mesh = plsc.VectorSubcoreMesh(core_axis_name="c", subcore_axis_name="s")
    return pl.kernel(
        functools.partial(_sc_gather_body, rpw, nchunk),
        out_type=jax.ShapeDtypeStruct((rows, D), jnp.float32),
        mesh=mesh,
        scratch_types=[
            pltpu.VMEM((rpw,), jnp.int32),
            pltpu.VMEM((NBUF, CH, D), jnp.float32),
            pltpu.SemaphoreType.DMA,
            pltpu.SemaphoreType.DMA,
            pltpu.SemaphoreType.DMA,
        ],
    )(emb, idx_flat)


def _tc_body(x_ref, wq_ref, w2c_ref, b1_ref, b2_ref, b3_ref,
             wd_ref, bd_ref, wlp_ref, blp_ref, out_ref):
    xb = x_ref[...].astype(jnp.bfloat16)
    xs = jnp.concatenate([xb[1:RB, :], xb[RB - 1:RB, :]], axis=0)
    xx = jnp.concatenate([xb, xs], axis=1)
    q2 = jnp.dot(xx, wq_ref[...], preferred_element_type=jnp.float32)
    q = q2.reshape(TB, L, 512)
    r2 = jnp.dot(xb, w2c_ref[...], preferred_element_type=jnp.float32)
    r3 = r2.reshape(TB, L, F)

    s1 = q[:, 0:199, 0:128] + b1_ref[...]
    s2 = q[:, 0:198, 128:256] + r3[:, 2:200, :] + b2_ref[...]
    s3 = (q[:, 0:197, 256:384] + q[:, 2:199, 384:512]
          + b3_ref[...])
    m1 = jnp.max(jnp.maximum(s1, 0.0), axis=1)
    m2 = jnp.max(jnp.maximum(s2, 0.0), axis=1)
    m3 = jnp.max(jnp.maximum(s3, 0.0), axis=1)
    cat = jnp.concatenate([m1, m2, m3], axis=1)

    h = jnp.maximum(
        jnp.dot(cat, wd_ref[...], preferred_element_type=jnp.float32)
        + bd_ref[...], 0.0)
    logits = (jnp.dot(h, wlp_ref[...], preferred_element_type=jnp.float32)
              + blp_ref[...])
    mx = jnp.max(logits, axis=1, keepdims=True)
    e = jnp.exp(logits - mx)
    out_ref[...] = e / jnp.sum(e, axis=1, keepdims=True)


def _tc_forward(x2d, wq, w2c, b1r, b2r, b3r, wd, bdr, wlp, blp):
    nb = x2d.shape[0] // RB
    return pl.pallas_call(
        _tc_body,
        grid=(nb,),
        in_specs=[
            pl.BlockSpec((RB, D), lambda i: (i, 0)),
            pl.BlockSpec((2 * D, 512), lambda i: (0, 0)),
            pl.BlockSpec((D, F), lambda i: (0, 0)),
            pl.BlockSpec((1, 1, F), lambda i: (0, 0, 0)),
            pl.BlockSpec((1, 1, F), lambda i: (0, 0, 0)),
            pl.BlockSpec((1, 1, F), lambda i: (0, 0, 0)),
            pl.BlockSpec((3 * F, U), lambda i: (0, 0)),
            pl.BlockSpec((1, U), lambda i: (0, 0)),
            pl.BlockSpec((U, 128), lambda i: (0, 0)),
            pl.BlockSpec((1, 128), lambda i: (0, 0)),
        ],
        out_specs=pl.BlockSpec((TB, 128), lambda i: (i, 0)),
        out_shape=jax.ShapeDtypeStruct((nb * TB, 128), jnp.float32),
    )(x2d, wq, w2c, b1r, b2r, b3r, wd, bdr, wlp, blp)


NSLICE = 2


def kernel(inputs, emb, W1, b1, W2, b2, W3, b3, Wd, bd, Wl, bl, training):
    idx_flat = inputs.reshape(B * L).astype(jnp.int32)

    wq = jnp.concatenate([
        W1.reshape(2 * D, F),
        W2[0:2].reshape(2 * D, F),
        W3[0:2].reshape(2 * D, F),
        W3[2:4].reshape(2 * D, F),
    ], axis=1).astype(jnp.bfloat16)
    w2c = W2[2].astype(jnp.bfloat16)

    b1r = b1.reshape(1, 1, F)
    b2r = b2.reshape(1, 1, F)
    b3r = b3.reshape(1, 1, F)
    bdr = bd.reshape(1, U)
    wlp = jnp.pad(Wl, ((0, 0), (0, 128 - NCLS)))
    blp = jnp.concatenate(
        [bl, jnp.full((128 - NCLS,), -1e30, dtype=jnp.float32)]).reshape(1, 128)

    srows = ROWS // NSLICE
    outs = []
    x_prev = None
    for s in range(NSLICE):
        idx_s = lax.slice(idx_flat, (s * srows,), ((s + 1) * srows,))
        if x_prev is not None:
            idx_s, _ = lax.optimization_barrier((idx_s, x_prev))
        x_s = _sc_gather(emb, idx_s)
        x_prev = x_s
        outs.append(_tc_forward(x_s, wq, w2c, b1r, b2r, b3r, Wd, bdr,
                                wlp, blp))
    out = outs[0] if NSLICE == 1 else jnp.concatenate(outs, axis=0)
    return out[:, :NCLS]
